# Initial kernel scaffold; baseline (speedup 1.0000x reference)
import jax
import jax.numpy as jnp
from jax.experimental import pallas as pl


def kernel(x, adj, W1, W2, W3, W4, W5, W6):
    raise NotImplementedError("write your pallas kernel here")


# speedup vs baseline: 1.1454x; 1.1454x over previous
import functools

import jax
import jax.numpy as jnp
from jax.experimental import pallas as pl
from jax.experimental.pallas import tpu as pltpu

N = 8192
BLK = 512


def _layer_body(feat_ref, w_ref, adj_ref, out_ref, s_ref, *, act):
    @pl.when(pl.program_id(0) == 0)
    def _():
        s = jnp.dot(feat_ref[...], w_ref[...], preferred_element_type=jnp.float32)
        if act:
            s = jnp.tanh(s)
        s_ref[...] = s

    out_ref[...] = jnp.dot(adj_ref[...], s_ref[...],
                           preferred_element_type=jnp.float32)


def _gcn_layer(feat, W, adj, act):
    n, f = feat.shape
    e = W.shape[1]
    return pl.pallas_call(
        functools.partial(_layer_body, act=act),
        grid=(n // BLK,),
        in_specs=[
            pl.BlockSpec((n, f), lambda i: (0, 0)),
            pl.BlockSpec((f, e), lambda i: (0, 0)),
            pl.BlockSpec((BLK, n), lambda i: (i, 0)),
        ],
        out_specs=pl.BlockSpec((BLK, e), lambda i: (i, 0)),
        out_shape=jax.ShapeDtypeStruct((n, e), jnp.float32),
        scratch_shapes=[pltpu.VMEM((n, e), jnp.float32)],
    )(feat, W, adj)


def _adjhat_body(zi_blk_ref, zh_blk_ref, zi_ref, zh_ref, out_ref):
    dn = (((1,), (1,)), ((), ()))
    a = jax.lax.dot_general(zi_blk_ref[...], zi_ref[...], dn,
                            preferred_element_type=jnp.float32)
    b = jax.lax.dot_general(zh_blk_ref[...], zh_ref[...], dn,
                            preferred_element_type=jnp.float32)
    out_ref[...] = jax.nn.sigmoid(a) + jax.nn.sigmoid(b)


def _adj_hat(z_igae, z_hat):
    n, e1 = z_igae.shape
    e2 = z_hat.shape[1]
    return pl.pallas_call(
        _adjhat_body,
        grid=(n // BLK,),
        in_specs=[
            pl.BlockSpec((BLK, e1), lambda i: (i, 0)),
            pl.BlockSpec((BLK, e2), lambda i: (i, 0)),
            pl.BlockSpec((n, e1), lambda i: (0, 0)),
            pl.BlockSpec((n, e2), lambda i: (0, 0)),
        ],
        out_specs=pl.BlockSpec((BLK, n), lambda i: (i, 0)),
        out_shape=jax.ShapeDtypeStruct((n, n), jnp.float32),
    )(z_igae, z_hat, z_igae, z_hat)


def kernel(x, adj, W1, W2, W3, W4, W5, W6):
    z1 = _gcn_layer(x, W1, adj, True)
    z2 = _gcn_layer(z1, W2, adj, True)
    z_igae = _gcn_layer(z2, W3, adj, False)
    zd1 = _gcn_layer(z_igae, W4, adj, True)
    zd2 = _gcn_layer(zd1, W5, adj, True)
    z_hat = _gcn_layer(zd2, W6, adj, True)
    adj_hat = _adj_hat(z_igae, z_hat)
    return (z_igae, z_hat, adj_hat)


# speedup vs baseline: 1.5175x; 1.3249x over previous
import functools

import jax
import jax.numpy as jnp
from jax.experimental import pallas as pl
from jax.experimental.pallas import tpu as pltpu

N = 8192
BLK = 512


def _layer1_body(feat_ref, w_ref, adj_ref, out_ref, adjh_ref, s_ref):
    @pl.when(pl.program_id(0) == 0)
    def _():
        s_ref[...] = jnp.tanh(
            jnp.dot(feat_ref[...], w_ref[...], preferred_element_type=jnp.float32))

    a = adj_ref[...]
    adjh_ref[...] = a.astype(jnp.bfloat16)
    out_ref[...] = jnp.dot(a, s_ref[...], preferred_element_type=jnp.float32)


def _layer1(feat, W, adj):
    n, f = feat.shape
    e = W.shape[1]
    return pl.pallas_call(
        _layer1_body,
        grid=(n // BLK,),
        in_specs=[
            pl.BlockSpec((n, f), lambda i: (0, 0)),
            pl.BlockSpec((f, e), lambda i: (0, 0)),
            pl.BlockSpec((BLK, n), lambda i: (i, 0)),
        ],
        out_specs=[
            pl.BlockSpec((BLK, e), lambda i: (i, 0)),
            pl.BlockSpec((BLK, n), lambda i: (i, 0)),
        ],
        out_shape=[
            jax.ShapeDtypeStruct((n, e), jnp.float32),
            jax.ShapeDtypeStruct((n, n), jnp.bfloat16),
        ],
        scratch_shapes=[pltpu.VMEM((n, e), jnp.float32)],
    )(feat, W, adj)


def _layer_body(feat_ref, w_ref, adj_ref, out_ref, s_ref, *, act):
    @pl.when(pl.program_id(0) == 0)
    def _():
        s = jnp.dot(feat_ref[...], w_ref[...], preferred_element_type=jnp.float32)
        if act:
            s = jnp.tanh(s)
        s_ref[...] = s

    out_ref[...] = jnp.dot(adj_ref[...], s_ref[...],
                           preferred_element_type=jnp.float32)


def _gcn_layer(feat, W, adjh, act):
    n, f = feat.shape
    e = W.shape[1]
    return pl.pallas_call(
        functools.partial(_layer_body, act=act),
        grid=(n // BLK,),
        in_specs=[
            pl.BlockSpec((n, f), lambda i: (0, 0)),
            pl.BlockSpec((f, e), lambda i: (0, 0)),
            pl.BlockSpec((BLK, n), lambda i: (i, 0)),
        ],
        out_specs=pl.BlockSpec((BLK, e), lambda i: (i, 0)),
        out_shape=jax.ShapeDtypeStruct((n, e), jnp.float32),
        scratch_shapes=[pltpu.VMEM((n, e), jnp.float32)],
    )(feat, W, adjh)


def _adjhat_body(zi_blk_ref, zh_blk_ref, zi_ref, zh_ref, out_ref):
    dn = (((1,), (1,)), ((), ()))
    a = jax.lax.dot_general(zi_blk_ref[...], zi_ref[...], dn,
                            preferred_element_type=jnp.float32)
    b = jax.lax.dot_general(zh_blk_ref[...], zh_ref[...], dn,
                            preferred_element_type=jnp.float32)
    out_ref[...] = (0.5 * jnp.tanh(0.5 * a)) + (0.5 * jnp.tanh(0.5 * b) + 1.0)


def _adj_hat(z_igae, z_hat):
    n, e1 = z_igae.shape
    e2 = z_hat.shape[1]
    return pl.pallas_call(
        _adjhat_body,
        grid=(n // BLK,),
        in_specs=[
            pl.BlockSpec((BLK, e1), lambda i: (i, 0)),
            pl.BlockSpec((BLK, e2), lambda i: (i, 0)),
            pl.BlockSpec((n, e1), lambda i: (0, 0)),
            pl.BlockSpec((n, e2), lambda i: (0, 0)),
        ],
        out_specs=pl.BlockSpec((BLK, n), lambda i: (i, 0)),
        out_shape=jax.ShapeDtypeStruct((n, n), jnp.float32),
    )(z_igae, z_hat, z_igae, z_hat)


def kernel(x, adj, W1, W2, W3, W4, W5, W6):
    z1, adjh = _layer1(x, W1, adj)
    z2 = _gcn_layer(z1, W2, adjh, True)
    z_igae = _gcn_layer(z2, W3, adjh, False)
    zd1 = _gcn_layer(z_igae, W4, adjh, True)
    zd2 = _gcn_layer(zd1, W5, adjh, True)
    z_hat = _gcn_layer(zd2, W6, adjh, True)
    adj_hat = _adj_hat(z_igae, z_hat)
    return (z_igae, z_hat, adj_hat)


# speedup vs baseline: 1.5681x; 1.0334x over previous
import functools

import jax
import jax.numpy as jnp
from jax.experimental import pallas as pl
from jax.experimental.pallas import tpu as pltpu

N = 8192
BLK = 512
E = 256


def _layer1_body(feat_ref, w_ref, adj_ref, out_ref, adjh_ref, s_ref):
    @pl.when(pl.program_id(0) == 0)
    def _():
        s_ref[...] = jnp.tanh(
            jnp.dot(feat_ref[...], w_ref[...], preferred_element_type=jnp.float32))

    a = adj_ref[...]
    adjh_ref[...] = a.astype(jnp.bfloat16)
    out_ref[...] = jnp.dot(a, s_ref[...], preferred_element_type=jnp.float32)


def _layer1(feat, W, adj):
    n, f = feat.shape
    e = W.shape[1]
    return pl.pallas_call(
        _layer1_body,
        grid=(n // BLK,),
        in_specs=[
            pl.BlockSpec((n, f), lambda i: (0, 0)),
            pl.BlockSpec((f, e), lambda i: (0, 0)),
            pl.BlockSpec((BLK, n), lambda i: (i, 0)),
        ],
        out_specs=[
            pl.BlockSpec((BLK, e), lambda i: (i, 0)),
            pl.BlockSpec((BLK, n), lambda i: (i, 0)),
        ],
        out_shape=[
            jax.ShapeDtypeStruct((n, e), jnp.float32),
            jax.ShapeDtypeStruct((n, n), jnp.bfloat16),
        ],
        scratch_shapes=[pltpu.VMEM((n, e), jnp.float32)],
    )(feat, W, adj)


def _chain_body(feat_ref, w_ref, adj_ref, out_ref, zprev_ref, s_ref, *,
                lin_layer):
    l = pl.program_id(0)
    i = pl.program_id(1)
    f = feat_ref.shape[1]

    @pl.when((l == 0) & (i == 0))
    def _():
        zprev_ref[:, :f] = feat_ref[...]
        if f < E:
            zprev_ref[:, f:] = jnp.zeros_like(zprev_ref[:, f:])

    @pl.when(i == 0)
    def _():
        s = jnp.dot(zprev_ref[...], w_ref[0], preferred_element_type=jnp.float32)
        if lin_layer >= 0:
            s = jnp.where(l == lin_layer, s, jnp.tanh(s))
        else:
            s = jnp.tanh(s)
        s_ref[...] = s.astype(jnp.bfloat16)

    z = jnp.dot(adj_ref[...], s_ref[...], preferred_element_type=jnp.float32)
    out_ref[...] = z
    zprev_ref[pl.ds(i * BLK, BLK), :] = z


def _layer_chain(feat, w_stack, adjh, lin_layer):
    n, f = feat.shape
    nl = w_stack.shape[0]
    return pl.pallas_call(
        functools.partial(_chain_body, lin_layer=lin_layer),
        grid=(nl, n // BLK),
        in_specs=[
            pl.BlockSpec((n, f), lambda l, i: (0, 0)),
            pl.BlockSpec((1, E, E), lambda l, i: (l, 0, 0)),
            pl.BlockSpec((BLK, n), lambda l, i: (i, 0)),
        ],
        out_specs=pl.BlockSpec((BLK, E), lambda l, i: (i, 0)),
        out_shape=jax.ShapeDtypeStruct((n, E), jnp.float32),
        scratch_shapes=[
            pltpu.VMEM((n, E), jnp.float32),
            pltpu.VMEM((n, E), jnp.bfloat16),
        ],
    )(feat, w_stack, adjh)


def _adjhat_body(zi_blk_ref, zh_blk_ref, zi_ref, zh_ref, out_ref):
    dn = (((1,), (1,)), ((), ()))
    a = jax.lax.dot_general(zi_blk_ref[...] * 0.5, zi_ref[...], dn,
                            preferred_element_type=jnp.float32)
    b = jax.lax.dot_general(zh_blk_ref[...] * 0.5, zh_ref[...], dn,
                            preferred_element_type=jnp.float32)
    out_ref[...] = (jnp.tanh(a) + jnp.tanh(b)) * 0.5 + 1.0


def _adj_hat(z_igae, z_hat):
    n, e1 = z_igae.shape
    e2 = z_hat.shape[1]
    return pl.pallas_call(
        _adjhat_body,
        grid=(n // BLK,),
        in_specs=[
            pl.BlockSpec((BLK, e1), lambda i: (i, 0)),
            pl.BlockSpec((BLK, e2), lambda i: (i, 0)),
            pl.BlockSpec((n, e1), lambda i: (0, 0)),
            pl.BlockSpec((n, e2), lambda i: (0, 0)),
        ],
        out_specs=pl.BlockSpec((BLK, n), lambda i: (i, 0)),
        out_shape=jax.ShapeDtypeStruct((n, n), jnp.float32),
    )(z_igae, z_hat, z_igae, z_hat)


def _pad_w(W):
    f, e = W.shape
    return jnp.pad(W, ((0, E - f), (0, E - e)))


def kernel(x, adj, W1, W2, W3, W4, W5, W6):
    z1, adjh = _layer1(x, W1, adj)
    w_enc = jnp.stack([_pad_w(W2), _pad_w(W3)])
    w_dec = jnp.stack([_pad_w(W4), _pad_w(W5), _pad_w(W6)])
    zi_pad = _layer_chain(z1, w_enc, adjh, lin_layer=1)
    zh_pad = _layer_chain(zi_pad, w_dec, adjh, lin_layer=-1)
    z_igae = zi_pad[:, :W3.shape[1]]
    z_hat = zh_pad[:, :W6.shape[1]]
    adj_hat = _adj_hat(z_igae, z_hat)
    return (z_igae, z_hat, adj_hat)


# speedup vs baseline: 1.6111x; 1.0274x over previous
import functools

import jax
import jax.numpy as jnp
from jax.experimental import pallas as pl
from jax.experimental.pallas import tpu as pltpu

N = 8192
BLK = 512
E = 256


def _layer1_body(feat_ref, w_ref, adj_ref, out_ref, adjh_ref, s_ref):
    @pl.when(pl.program_id(0) == 0)
    def _():
        s_ref[...] = jnp.tanh(
            jnp.dot(feat_ref[...], w_ref[...], preferred_element_type=jnp.float32))

    a = adj_ref[...]
    adjh_ref[...] = a.astype(jnp.bfloat16)
    out_ref[...] = jnp.dot(a, s_ref[...], preferred_element_type=jnp.float32)


def _layer1(feat, W, adj):
    n, f = feat.shape
    e = W.shape[1]
    return pl.pallas_call(
        _layer1_body,
        grid=(n // BLK,),
        in_specs=[
            pl.BlockSpec((n, f), lambda i: (0, 0)),
            pl.BlockSpec((f, e), lambda i: (0, 0)),
            pl.BlockSpec((BLK, n), lambda i: (i, 0)),
        ],
        out_specs=[
            pl.BlockSpec((BLK, e), lambda i: (i, 0)),
            pl.BlockSpec((BLK, n), lambda i: (i, 0)),
        ],
        out_shape=[
            jax.ShapeDtypeStruct((n, e), jnp.float32),
            jax.ShapeDtypeStruct((n, n), jnp.bfloat16),
        ],
        scratch_shapes=[pltpu.VMEM((n, e), jnp.float32)],
    )(feat, W, adj)


def _chain_body(feat_ref, w_ref, adj_ref, zi_ref, zh_ref,
                zprev_ref, s_ref, zi_keep_ref, *, lin_layer, e_zi, e_zh):
    l = pl.program_id(0)
    i = pl.program_id(1)
    f = feat_ref.shape[1]

    @pl.when((l == 0) & (i == 0))
    def _():
        zprev_ref[:, :f] = feat_ref[...]
        if f < E:
            zprev_ref[:, f:] = jnp.zeros_like(zprev_ref[:, f:])

    @pl.when(i == 0)
    def _():
        s = jnp.dot(zprev_ref[...], w_ref[0], preferred_element_type=jnp.float32)
        s = jnp.where(l == lin_layer, s, jnp.tanh(s))
        s_ref[...] = s.astype(jnp.bfloat16)

    z = jnp.dot(adj_ref[...], s_ref[...], preferred_element_type=jnp.float32)
    zprev_ref[pl.ds(i * BLK, BLK), :] = z

    @pl.when(l == 1)
    def _():
        zi_keep_ref[pl.ds(i * BLK, BLK), :] = z[:, :e_zi]

    zi_ref[...] = zi_keep_ref[pl.ds(i * BLK, BLK), :]
    zh_ref[...] = z[:, :e_zh]


def _layer_chain(feat, w_stack, adjh, lin_layer, e_zi, e_zh):
    n, f = feat.shape
    nl = w_stack.shape[0]
    return pl.pallas_call(
        functools.partial(_chain_body, lin_layer=lin_layer, e_zi=e_zi, e_zh=e_zh),
        grid=(nl, n // BLK),
        in_specs=[
            pl.BlockSpec((n, f), lambda l, i: (0, 0)),
            pl.BlockSpec((1, E, E), lambda l, i: (l, 0, 0)),
            pl.BlockSpec((BLK, n), lambda l, i: (i, 0)),
        ],
        out_specs=[
            pl.BlockSpec((BLK, e_zi), lambda l, i: (i, 0)),
            pl.BlockSpec((BLK, e_zh), lambda l, i: (i, 0)),
        ],
        out_shape=[
            jax.ShapeDtypeStruct((n, e_zi), jnp.float32),
            jax.ShapeDtypeStruct((n, e_zh), jnp.float32),
        ],
        scratch_shapes=[
            pltpu.VMEM((n, E), jnp.float32),
            pltpu.VMEM((n, E), jnp.bfloat16),
            pltpu.VMEM((n, e_zi), jnp.float32),
        ],
    )(feat, w_stack, adjh)


def _adjhat_body(zi_blk_ref, zh_blk_ref, zi_ref, zh_ref, out_ref):
    dn = (((1,), (1,)), ((), ()))
    a = jax.lax.dot_general(zi_blk_ref[...] * 0.5, zi_ref[...], dn,
                            preferred_element_type=jnp.float32)
    b = jax.lax.dot_general(zh_blk_ref[...] * 0.5, zh_ref[...], dn,
                            preferred_element_type=jnp.float32)
    out_ref[...] = (jnp.tanh(a) + jnp.tanh(b)) * 0.5 + 1.0


def _adj_hat(z_igae, z_hat):
    n, e1 = z_igae.shape
    e2 = z_hat.shape[1]
    return pl.pallas_call(
        _adjhat_body,
        grid=(n // BLK,),
        in_specs=[
            pl.BlockSpec((BLK, e1), lambda i: (i, 0)),
            pl.BlockSpec((BLK, e2), lambda i: (i, 0)),
            pl.BlockSpec((n, e1), lambda i: (0, 0)),
            pl.BlockSpec((n, e2), lambda i: (0, 0)),
        ],
        out_specs=pl.BlockSpec((BLK, n), lambda i: (i, 0)),
        out_shape=jax.ShapeDtypeStruct((n, n), jnp.float32),
    )(z_igae, z_hat, z_igae, z_hat)


def _pad_w(W):
    f, e = W.shape
    return jnp.pad(W, ((0, E - f), (0, E - e)))


def kernel(x, adj, W1, W2, W3, W4, W5, W6):
    z1, adjh = _layer1(x, W1, adj)
    w_stack = jnp.stack([_pad_w(W2), _pad_w(W3), _pad_w(W4), _pad_w(W5),
                         _pad_w(W6)])
    z_igae, z_hat = _layer_chain(z1, w_stack, adjh, lin_layer=1,
                                 e_zi=W3.shape[1], e_zh=W6.shape[1])
    adj_hat = _adj_hat(z_igae, z_hat)
    return (z_igae, z_hat, adj_hat)


# speedup vs baseline: 1.7002x; 1.0553x over previous
import functools

import jax
import jax.numpy as jnp
from jax.experimental import pallas as pl
from jax.experimental.pallas import tpu as pltpu

N = 8192
BLK = 512
CHAIN_BLK = 1024
E = 256


def _layer1_body(feat_ref, w_ref, adj_ref, out_ref, adjh_ref, s_ref):
    @pl.when(pl.program_id(0) == 0)
    def _():
        s_ref[...] = jnp.tanh(
            jnp.dot(feat_ref[...], w_ref[...], preferred_element_type=jnp.float32))

    a = adj_ref[...]
    adjh_ref[...] = a.astype(jnp.bfloat16)
    out_ref[...] = jnp.dot(a, s_ref[...], preferred_element_type=jnp.float32)


def _layer1(feat, W, adj):
    n, f = feat.shape
    e = W.shape[1]
    return pl.pallas_call(
        _layer1_body,
        grid=(n // BLK,),
        in_specs=[
            pl.BlockSpec((n, f), lambda i: (0, 0)),
            pl.BlockSpec((f, e), lambda i: (0, 0)),
            pl.BlockSpec((BLK, n), lambda i: (i, 0)),
        ],
        out_specs=[
            pl.BlockSpec((BLK, e), lambda i: (i, 0)),
            pl.BlockSpec((BLK, n), lambda i: (i, 0)),
        ],
        out_shape=[
            jax.ShapeDtypeStruct((n, e), jnp.float32),
            jax.ShapeDtypeStruct((n, n), jnp.bfloat16),
        ],
        scratch_shapes=[pltpu.VMEM((n, e), jnp.float32)],
    )(feat, W, adj)


def _chain_body(feat_ref, w_ref, adj_ref, zi_ref, zh_ref,
                zprev_ref, s_ref, zi_keep_ref, *, lin_layer, e_zi, e_zh):
    l = pl.program_id(0)
    i = pl.program_id(1)
    f = feat_ref.shape[1]
    blk = adj_ref.shape[0]

    @pl.when((l == 0) & (i == 0))
    def _():
        zprev_ref[:, :f] = feat_ref[...]
        if f < E:
            zprev_ref[:, f:] = jnp.zeros_like(zprev_ref[:, f:])

    @pl.when(i == 0)
    def _():
        s = jnp.dot(zprev_ref[...], w_ref[0], preferred_element_type=jnp.float32)
        s = jnp.where(l == lin_layer, s, jnp.tanh(s))
        s_ref[...] = s.astype(jnp.bfloat16)

    z = jnp.dot(adj_ref[...], s_ref[...], preferred_element_type=jnp.float32)
    zprev_ref[pl.ds(i * blk, blk), :] = z

    @pl.when(l == 1)
    def _():
        zi_keep_ref[pl.ds(i * blk, blk), :] = z[:, :e_zi]

    zi_ref[...] = zi_keep_ref[pl.ds(i * blk, blk), :]
    zh_ref[...] = z[:, :e_zh]


def _layer_chain(feat, w_stack, adjh, lin_layer, e_zi, e_zh):
    n, f = feat.shape
    nl = w_stack.shape[0]
    return pl.pallas_call(
        functools.partial(_chain_body, lin_layer=lin_layer, e_zi=e_zi, e_zh=e_zh),
        grid=(nl, n // CHAIN_BLK),
        in_specs=[
            pl.BlockSpec((n, f), lambda l, i: (0, 0)),
            pl.BlockSpec((1, E, E), lambda l, i: (l, 0, 0)),
            pl.BlockSpec((CHAIN_BLK, n), lambda l, i: (i, 0)),
        ],
        out_specs=[
            pl.BlockSpec((CHAIN_BLK, e_zi), lambda l, i: (i, 0)),
            pl.BlockSpec((CHAIN_BLK, e_zh), lambda l, i: (i, 0)),
        ],
        out_shape=[
            jax.ShapeDtypeStruct((n, e_zi), jnp.float32),
            jax.ShapeDtypeStruct((n, e_zh), jnp.float32),
        ],
        scratch_shapes=[
            pltpu.VMEM((n, E), jnp.float32),
            pltpu.VMEM((n, E), jnp.bfloat16),
            pltpu.VMEM((n, e_zi), jnp.float32),
        ],
    )(feat, w_stack, adjh)


def _adjhat_body(zi_blk_ref, zh_blk_ref, zi_ref, zh_ref, out_ref):
    dn = (((1,), (1,)), ((), ()))
    a = jax.lax.dot_general(zi_blk_ref[...] * 0.5, zi_ref[...], dn,
                            preferred_element_type=jnp.float32)
    b = jax.lax.dot_general(zh_blk_ref[...] * 0.5, zh_ref[...], dn,
                            preferred_element_type=jnp.float32)
    out_ref[...] = (jnp.tanh(a) + jnp.tanh(b)) * 0.5 + 1.0


def _adj_hat(z_igae, z_hat):
    n, e1 = z_igae.shape
    e2 = z_hat.shape[1]
    return pl.pallas_call(
        _adjhat_body,
        grid=(n // BLK,),
        in_specs=[
            pl.BlockSpec((BLK, e1), lambda i: (i, 0)),
            pl.BlockSpec((BLK, e2), lambda i: (i, 0)),
            pl.BlockSpec((n, e1), lambda i: (0, 0)),
            pl.BlockSpec((n, e2), lambda i: (0, 0)),
        ],
        out_specs=pl.BlockSpec((BLK, n), lambda i: (i, 0)),
        out_shape=jax.ShapeDtypeStruct((n, n), jnp.float32),
    )(z_igae, z_hat, z_igae, z_hat)


def _pad_w(W):
    f, e = W.shape
    return jnp.pad(W, ((0, E - f), (0, E - e)))


def kernel(x, adj, W1, W2, W3, W4, W5, W6):
    z1, adjh = _layer1(x, W1, adj)
    w_stack = jnp.stack([_pad_w(W2), _pad_w(W3), _pad_w(W4), _pad_w(W5),
                         _pad_w(W6)])
    z_igae, z_hat = _layer_chain(z1, w_stack, adjh, lin_layer=1,
                                 e_zi=W3.shape[1], e_zh=W6.shape[1])
    adj_hat = _adj_hat(z_igae, z_hat)
    return (z_igae, z_hat, adj_hat)
